# baseline (speedup 1.0000x reference)
import functools

import jax
import jax.numpy as jnp
from jax import lax
from jax.experimental import pallas as pl
from jax.experimental.pallas import tpu as pltpu
from jax.experimental.pallas import tpu_sc as plsc

N_NODES = 10000
N_EDGES = 320000
T_DIM = 64
L_LAYERS = 4
LANES = 16
N_WORKERS = 32
CHUNK = 8000
N_CHUNKS = N_EDGES // CHUNK
EDGES_PER_W = N_EDGES // N_WORKERS

_mesh = plsc.VectorSubcoreMesh(core_axis_name="c", subcore_axis_name="s")
_sc_params = pltpu.CompilerParams(needs_layout_passes=False)


def _wid():
    return lax.axis_index("s") * 2 + lax.axis_index("c")


def _zero_f32(ref, n_words):
    z = jnp.zeros((LANES,), jnp.float32)

    @plsc.parallel_loop(0, n_words // LANES, unroll=8)
    def body(i):
        ref[pl.ds(i * LANES, LANES)] = z


@functools.partial(
    pl.kernel,
    out_type=jax.ShapeDtypeStruct((N_WORKERS * N_NODES,), jnp.float32),
    mesh=_mesh,
    compiler_params=_sc_params,
    scratch_types=[
        pltpu.VMEM((EDGES_PER_W,), jnp.int32),
        pltpu.VMEM((N_NODES,), jnp.float32),
    ],
)
def _deg_kernel(epk_hbm, out_hbm, idx_v, deg_v):
    w = _wid()
    pltpu.sync_copy(epk_hbm.at[pl.ds(w * EDGES_PER_W, EDGES_PER_W)], idx_v)
    _zero_f32(deg_v, N_NODES)
    ones = jnp.ones((LANES,), jnp.float32)

    @plsc.parallel_loop(0, EDGES_PER_W // LANES, unroll=8)
    def body(b):
        pk = idx_v[pl.ds(b * LANES, LANES)]
        srcv = jnp.bitwise_and(pk, 0xFFFF)
        plsc.addupdate_scatter(deg_v, [srcv], ones)
    pltpu.sync_copy(deg_v, out_hbm.at[pl.ds(w * N_NODES, N_NODES)])


def _coeff_body(a1_ref, g_ref, b_ref, degp_ref, A_ref, B_ref, biasb_ref):
    deg = jnp.sum(degp_ref[...], axis=0, keepdims=True)
    ldeg = jnp.log(deg)
    for i in range(L_LAYERS):
        a1 = a1_ref[i]
        dp = jax.nn.sigmoid(g_ref[i])
        sw = jnp.exp(a1)
        nw = sw * jnp.tanh(a1)
        A_ref[pl.ds(i, 1), :] = sw * jnp.exp(dp * ldeg)
        B_ref[pl.ds(i, 1), :] = nw * jnp.exp((dp - 1.0) * ldeg)
        biasb_ref[pl.ds(i, 1), :] = jnp.full((1, 128), b_ref[i], jnp.float32)


def _coeff_call(a1, g, b, degp):
    return pl.pallas_call(
        _coeff_body,
        out_shape=(
            jax.ShapeDtypeStruct((L_LAYERS, N_NODES), jnp.float32),
            jax.ShapeDtypeStruct((L_LAYERS, N_NODES), jnp.float32),
            jax.ShapeDtypeStruct((L_LAYERS, 128), jnp.float32),
        ),
        in_specs=[
            pl.BlockSpec(memory_space=pltpu.SMEM),
            pl.BlockSpec(memory_space=pltpu.SMEM),
            pl.BlockSpec(memory_space=pltpu.SMEM),
            pl.BlockSpec(memory_space=pltpu.VMEM),
        ],
    )(a1, g, b, degp)


@functools.partial(
    pl.kernel,
    out_type=jax.ShapeDtypeStruct((T_DIM * N_NODES,), jnp.float32),
    mesh=_mesh,
    compiler_params=_sc_params,
    scratch_types=[
        pltpu.VMEM((N_NODES,), jnp.float32),
        pltpu.VMEM((N_NODES,), jnp.float32),
        pltpu.VMEM((N_NODES,), jnp.float32),
        pltpu.VMEM((N_NODES,), jnp.float32),
        pltpu.VMEM((N_NODES,), jnp.float32),
        pltpu.VMEM((N_NODES,), jnp.float32),
        pltpu.VMEM((N_NODES,), jnp.int32),
        pltpu.VMEM((CHUNK,), jnp.int32),
        pltpu.VMEM((CHUNK,), jnp.int32),
        pltpu.VMEM((L_LAYERS * 128,), jnp.float32),
        pltpu.SemaphoreType.DMA,
        pltpu.SemaphoreType.DMA,
        pltpu.SemaphoreType.DMA,
        pltpu.SemaphoreType.DMA,
    ],
)
def _main_kernel(x_hbm, epk_hbm, A_hbm, B_hbm, biasb_hbm, out_hbm,
                 x0, x1, S0, S1, Ab, Bb, xp, eb0, eb1, bb, se0, se1, sA, sB):
    w = _wid()
    r0 = (2 * w) * N_NODES
    r1 = r0 + N_NODES
    pltpu.sync_copy(x_hbm.at[pl.ds(r0, N_NODES)], x0)
    pltpu.sync_copy(x_hbm.at[pl.ds(r1, N_NODES)], x1)
    pltpu.sync_copy(biasb_hbm, bb)

    def pack_cols(a, b):
        return plsc.bitcast(
            plsc.pack(a, b, format=plsc.PackFormat.INTERLEAVED), jnp.int32)

    @plsc.parallel_loop(0, N_NODES // LANES, unroll=8)
    def initpack(n):
        sl = pl.ds(n * LANES, LANES)
        xp[sl] = pack_cols(x0[sl], x1[sl])

    def edge_start(g, buf, sem):
        pltpu.make_async_copy(epk_hbm.at[pl.ds(g * CHUNK, CHUNK)], buf, sem).start()

    def edge_wait(buf, sem):
        pltpu.make_async_copy(epk_hbm.at[pl.ds(0, CHUNK)], buf, sem).wait()

    def process(buf):
        @plsc.parallel_loop(0, CHUNK // LANES, unroll=24)
        def body(b):
            pk = buf[pl.ds(b * LANES, LANES)]
            srcv = jnp.bitwise_and(pk, 0xFFFF)
            dstv = lax.shift_right_logical(pk, 16)
            g = plsc.load_gather(xp, [srcv])
            g0, g1 = plsc.unpack(
                plsc.bitcast(g, jnp.bfloat16),
                format=plsc.PackFormat.INTERLEAVED,
                preferred_element_type=jnp.float32)
            plsc.addupdate_scatter(S0, [dstv], g0)
            plsc.addupdate_scatter(S1, [dstv], g1)

    def layer(i, _):
        cpA = pltpu.make_async_copy(A_hbm.at[pl.ds(i * N_NODES, N_NODES)], Ab, sA)
        cpB = pltpu.make_async_copy(B_hbm.at[pl.ds(i * N_NODES, N_NODES)], Bb, sB)
        cpA.start()
        cpB.start()
        _zero_f32(S0, N_NODES)
        _zero_f32(S1, N_NODES)
        edge_start(0, eb0, se0)
        edge_start(1, eb1, se1)

        def chunk2(k, _):
            edge_wait(eb0, se0)
            process(eb0)

            @pl.when(2 * k + 2 < N_CHUNKS)
            def _():
                edge_start(2 * k + 2, eb0, se0)

            edge_wait(eb1, se1)
            process(eb1)

            @pl.when(2 * k + 3 < N_CHUNKS)
            def _():
                edge_start(2 * k + 3, eb1, se1)

            return None

        lax.fori_loop(0, N_CHUNKS // 2, chunk2, None)
        cpA.wait()
        cpB.wait()
        bias_v = bb[pl.ds(i * 128, LANES)]

        @plsc.parallel_loop(0, N_NODES // LANES, unroll=8)
        def combine(n):
            sl = pl.ds(n * LANES, LANES)
            a = Ab[sl]
            bcoef = Bb[sl]
            nx0 = a * x0[sl] + bcoef * S0[sl] + bias_v
            nx1 = a * x1[sl] + bcoef * S1[sl] + bias_v
            x0[sl] = nx0
            x1[sl] = nx1
            xp[sl] = pack_cols(nx0, nx1)

        return None

    lax.fori_loop(0, L_LAYERS, layer, None)
    pltpu.sync_copy(x0, out_hbm.at[pl.ds(r0, N_NODES)])
    pltpu.sync_copy(x1, out_hbm.at[pl.ds(r1, N_NODES)])


def kernel(x, edge_index, alpha1, alpha2, gamma, bias):
    del alpha2
    src = edge_index[0].astype(jnp.int32)
    dst = edge_index[1].astype(jnp.int32)
    epk = jnp.bitwise_or(src, lax.shift_left(dst, 16))
    degp = _deg_kernel(epk).reshape(N_WORKERS, N_NODES)
    A, B, biasb = _coeff_call(
        alpha1.reshape(L_LAYERS), gamma.reshape(L_LAYERS),
        bias.reshape(L_LAYERS), degp)
    out = _main_kernel(
        x.reshape(T_DIM * N_NODES), epk,
        A.reshape(L_LAYERS * N_NODES), B.reshape(L_LAYERS * N_NODES),
        biasb.reshape(L_LAYERS * 128))
    return out.reshape(T_DIM, N_NODES)


# speedup vs baseline: 1.2790x; 1.2790x over previous
import functools

import jax
import jax.numpy as jnp
from jax import lax
from jax.experimental import pallas as pl
from jax.experimental.pallas import tpu as pltpu
from jax.experimental.pallas import tpu_sc as plsc

N_NODES = 10000
N_EDGES = 320000
T_DIM = 64
L_LAYERS = 4
LANES = 16
N_WORKERS = 32
N_PAD = N_NODES + LANES
EDGES_PER_W = N_EDGES // N_WORKERS
CAP = 720
BIN_WORDS = LANES * CAP
EBIN_TOTAL = N_WORKERS * BIN_WORDS
SPILL_WORDS = EDGES_PER_W + LANES
ESPILL_TOTAL = N_WORKERS * SPILL_WORDS
CHUNK = 9216
N_CHUNKS = EBIN_TOTAL // CHUNK

_mesh = plsc.VectorSubcoreMesh(core_axis_name="c", subcore_axis_name="s")
_sc_params = pltpu.CompilerParams(needs_layout_passes=False)


def _wid():
    return lax.axis_index("s") * 2 + lax.axis_index("c")


def _zero_f32(ref, n_words):
    z = jnp.zeros((LANES,), jnp.float32)

    @plsc.parallel_loop(0, n_words // LANES, unroll=8)
    def body(i):
        ref[pl.ds(i * LANES, LANES)] = z


def _sentinel_vec():
    lane = lax.iota(jnp.int32, LANES)
    node = lane + N_NODES
    return jnp.bitwise_or(node, lax.shift_left(node, 16))


@functools.partial(
    pl.kernel,
    out_type=(
        jax.ShapeDtypeStruct((N_WORKERS * N_NODES,), jnp.float32),
        jax.ShapeDtypeStruct((EBIN_TOTAL,), jnp.int32),
        jax.ShapeDtypeStruct((ESPILL_TOTAL,), jnp.int32),
        jax.ShapeDtypeStruct((N_WORKERS * LANES,), jnp.int32),
    ),
    mesh=_mesh,
    compiler_params=_sc_params,
    scratch_types=[
        pltpu.VMEM((EDGES_PER_W,), jnp.int32),
        pltpu.VMEM((N_NODES,), jnp.float32),
        pltpu.VMEM((BIN_WORDS,), jnp.int32),
        pltpu.VMEM((SPILL_WORDS,), jnp.int32),
        pltpu.VMEM((LANES,), jnp.int32),
        pltpu.VMEM((LANES,), jnp.int32),
        pltpu.VMEM((LANES,), jnp.int32),
    ],
)
def _bin_kernel(epk_hbm, degp_hbm, ebin_hbm, espill_hbm, ehdr_hbm,
                idx_v, deg_v, bin_v, sp_v, cnt_v, spc_v, hdr_v):
    w = _wid()
    pltpu.sync_copy(epk_hbm.at[pl.ds(w * EDGES_PER_W, EDGES_PER_W)], idx_v)
    _zero_f32(deg_v, N_NODES)
    sent = _sentinel_vec()

    @plsc.parallel_loop(0, CAP, unroll=8)
    def fill_bin(r):
        bin_v[pl.ds(r * LANES, LANES)] = sent

    @plsc.parallel_loop(0, SPILL_WORDS // LANES, unroll=8)
    def fill_sp(r):
        sp_v[pl.ds(r * LANES, LANES)] = sent

    zero_i = jnp.zeros((LANES,), jnp.int32)
    ones_i = jnp.ones((LANES,), jnp.int32)
    ones_f = jnp.ones((LANES,), jnp.float32)
    cnt_v[...] = zero_i
    spc_v[...] = zero_i

    def bbody(b, _):
        pk = idx_v[pl.ds(b * LANES, LANES)]
        srcv = jnp.bitwise_and(pk, 0xFFFF)
        dstv = lax.shift_right_logical(pk, 16)
        plsc.addupdate_scatter(deg_v, [srcv], ones_f)
        dig = jnp.bitwise_and(dstv, LANES - 1)
        base = plsc.load_gather(cnt_v, [dig])
        rank, _ = plsc.scan_count(dig)
        plsc.addupdate_scatter(cnt_v, [dig], ones_i)
        pos = base + rank
        okm = pos < CAP
        plsc.store_scatter(bin_v, [pos * LANES + dig], pk, mask=okm)
        spm = jnp.logical_not(okm)
        sbase = plsc.load_gather(spc_v, [zero_i])
        srank, _ = plsc.scan_count(zero_i, mask=spm)
        plsc.store_scatter(sp_v, [sbase + srank], pk, mask=spm)
        plsc.addupdate_scatter(spc_v, [zero_i], ones_i, mask=spm)
        return None

    lax.fori_loop(0, EDGES_PER_W // LANES, bbody, None)

    sct = jnp.max(spc_v[...])
    rounds = jnp.where(sct > 0, (sct + LANES) // LANES, 0)
    hdr_v[...] = jnp.full((LANES,), rounds, jnp.int32)
    pltpu.sync_copy(deg_v, degp_hbm.at[pl.ds(w * N_NODES, N_NODES)])
    pltpu.sync_copy(bin_v, ebin_hbm.at[pl.ds(w * BIN_WORDS, BIN_WORDS)])
    pltpu.sync_copy(sp_v, espill_hbm.at[pl.ds(w * SPILL_WORDS, SPILL_WORDS)])
    pltpu.sync_copy(hdr_v, ehdr_hbm.at[pl.ds(w * LANES, LANES)])


def _coeff_body(a1_ref, g_ref, b_ref, degp_ref, A_ref, B_ref, biasb_ref):
    deg = jnp.sum(degp_ref[...], axis=0, keepdims=True)
    ldeg = jnp.log(deg)
    for i in range(L_LAYERS):
        a1 = a1_ref[i]
        dp = jax.nn.sigmoid(g_ref[i])
        sw = jnp.exp(a1)
        nw = sw * jnp.tanh(a1)
        A_ref[pl.ds(i, 1), :] = sw * jnp.exp(dp * ldeg)
        B_ref[pl.ds(i, 1), :] = nw * jnp.exp((dp - 1.0) * ldeg)
        biasb_ref[pl.ds(i, 1), :] = jnp.full((1, 128), b_ref[i], jnp.float32)


def _coeff_call(a1, g, b, degp):
    return pl.pallas_call(
        _coeff_body,
        out_shape=(
            jax.ShapeDtypeStruct((L_LAYERS, N_NODES), jnp.float32),
            jax.ShapeDtypeStruct((L_LAYERS, N_NODES), jnp.float32),
            jax.ShapeDtypeStruct((L_LAYERS, 128), jnp.float32),
        ),
        in_specs=[
            pl.BlockSpec(memory_space=pltpu.SMEM),
            pl.BlockSpec(memory_space=pltpu.SMEM),
            pl.BlockSpec(memory_space=pltpu.SMEM),
            pl.BlockSpec(memory_space=pltpu.VMEM),
        ],
    )(a1, g, b, degp)


@functools.partial(
    pl.kernel,
    out_type=jax.ShapeDtypeStruct((T_DIM * N_NODES,), jnp.float32),
    mesh=_mesh,
    compiler_params=_sc_params,
    scratch_types=[
        pltpu.VMEM((N_PAD,), jnp.float32),
        pltpu.VMEM((N_PAD,), jnp.float32),
        pltpu.VMEM((N_PAD,), jnp.float32),
        pltpu.VMEM((N_PAD,), jnp.float32),
        pltpu.VMEM((N_NODES,), jnp.float32),
        pltpu.VMEM((N_NODES,), jnp.float32),
        pltpu.VMEM((N_PAD,), jnp.int32),
        pltpu.VMEM((CHUNK,), jnp.int32),
        pltpu.VMEM((CHUNK,), jnp.int32),
        pltpu.VMEM((L_LAYERS * 128,), jnp.float32),
        pltpu.VMEM((N_WORKERS * LANES,), jnp.int32),
        pltpu.VMEM((LANES,), jnp.int32),
        pltpu.SemaphoreType.DMA,
        pltpu.SemaphoreType.DMA,
        pltpu.SemaphoreType.DMA,
        pltpu.SemaphoreType.DMA,
    ],
)
def _main_kernel(x_hbm, ebin_hbm, espill_hbm, ehdr_hbm, A_hbm, B_hbm,
                 biasb_hbm, out_hbm,
                 x0, x1, S0, S1, Ab, Bb, xp, eb0, eb1, bb, hdrv, sbuf,
                 se0, se1, sA, sB):
    w = _wid()
    r0 = (2 * w) * N_NODES
    r1 = r0 + N_NODES
    pltpu.sync_copy(x_hbm.at[pl.ds(r0, N_NODES)], x0.at[pl.ds(0, N_NODES)])
    pltpu.sync_copy(x_hbm.at[pl.ds(r1, N_NODES)], x1.at[pl.ds(0, N_NODES)])
    pltpu.sync_copy(biasb_hbm, bb)
    pltpu.sync_copy(ehdr_hbm, hdrv)
    zpad = jnp.zeros((LANES,), jnp.float32)
    x0[pl.ds(N_NODES, LANES)] = zpad
    x1[pl.ds(N_NODES, LANES)] = zpad

    def pack_cols(a, b):
        return plsc.bitcast(
            plsc.pack(a, b, format=plsc.PackFormat.INTERLEAVED), jnp.int32)

    @plsc.parallel_loop(0, N_PAD // LANES, unroll=8)
    def initpack(n):
        sl = pl.ds(n * LANES, LANES)
        xp[sl] = pack_cols(x0[sl], x1[sl])

    def edge_start(g, buf, sem):
        pltpu.make_async_copy(ebin_hbm.at[pl.ds(g * CHUNK, CHUNK)], buf, sem).start()

    def edge_wait(buf, sem):
        pltpu.make_async_copy(ebin_hbm.at[pl.ds(0, CHUNK)], buf, sem).wait()

    def scatter_batch(pk):
        srcv = jnp.bitwise_and(pk, 0xFFFF)
        dstv = lax.shift_right_logical(pk, 16)
        g = plsc.load_gather(xp, [srcv])
        g0, g1 = plsc.unpack(
            plsc.bitcast(g, jnp.bfloat16),
            format=plsc.PackFormat.INTERLEAVED,
            preferred_element_type=jnp.float32)
        plsc.addupdate_scatter(S0, [dstv], g0)
        plsc.addupdate_scatter(S1, [dstv], g1)

    def process(buf):
        @plsc.parallel_loop(0, CHUNK // LANES, unroll=16)
        def body(b):
            scatter_batch(buf[pl.ds(b * LANES, LANES)])

    def layer(i, _):
        cpA = pltpu.make_async_copy(A_hbm.at[pl.ds(i * N_NODES, N_NODES)], Ab, sA)
        cpB = pltpu.make_async_copy(B_hbm.at[pl.ds(i * N_NODES, N_NODES)], Bb, sB)
        cpA.start()
        cpB.start()
        _zero_f32(S0, N_PAD)
        _zero_f32(S1, N_PAD)
        edge_start(0, eb0, se0)
        edge_start(1, eb1, se1)

        def chunk2(k, _):
            edge_wait(eb0, se0)
            process(eb0)

            @pl.when(2 * k + 2 < N_CHUNKS)
            def _():
                edge_start(2 * k + 2, eb0, se0)

            edge_wait(eb1, se1)
            process(eb1)

            @pl.when(2 * k + 3 < N_CHUNKS)
            def _():
                edge_start(2 * k + 3, eb1, se1)

            return None

        lax.fori_loop(0, N_CHUNKS // 2, chunk2, None)

        def spill_chunk(c, _):
            sr = jnp.max(hdrv[pl.ds(c * LANES, LANES)])

            @pl.when(sr > 0)
            def _():
                def srow(r, _2):
                    pltpu.sync_copy(
                        espill_hbm.at[pl.ds(c * SPILL_WORDS + r * LANES, LANES)],
                        sbuf)
                    scatter_batch(sbuf[...])
                    return None

                lax.fori_loop(0, sr, srow, None)

            return None

        lax.fori_loop(0, N_WORKERS, spill_chunk, None)

        cpA.wait()
        cpB.wait()
        bias_v = bb[pl.ds(i * 128, LANES)]

        @plsc.parallel_loop(0, N_NODES // LANES, unroll=8)
        def combine(n):
            sl = pl.ds(n * LANES, LANES)
            a = Ab[sl]
            bcoef = Bb[sl]
            nx0 = a * x0[sl] + bcoef * S0[sl] + bias_v
            nx1 = a * x1[sl] + bcoef * S1[sl] + bias_v
            x0[sl] = nx0
            x1[sl] = nx1
            xp[sl] = pack_cols(nx0, nx1)

        return None

    lax.fori_loop(0, L_LAYERS, layer, None)
    pltpu.sync_copy(x0.at[pl.ds(0, N_NODES)], out_hbm.at[pl.ds(r0, N_NODES)])
    pltpu.sync_copy(x1.at[pl.ds(0, N_NODES)], out_hbm.at[pl.ds(r1, N_NODES)])


def kernel(x, edge_index, alpha1, alpha2, gamma, bias):
    del alpha2
    src = edge_index[0].astype(jnp.int32)
    dst = edge_index[1].astype(jnp.int32)
    epk = jnp.bitwise_or(src, lax.shift_left(dst, 16))
    degp, ebin, espill, ehdr = _bin_kernel(epk)
    A, B, biasb = _coeff_call(
        alpha1.reshape(L_LAYERS), gamma.reshape(L_LAYERS),
        bias.reshape(L_LAYERS), degp.reshape(N_WORKERS, N_NODES))
    out = _main_kernel(
        x.reshape(T_DIM * N_NODES), ebin, espill, ehdr,
        A.reshape(L_LAYERS * N_NODES), B.reshape(L_LAYERS * N_NODES),
        biasb.reshape(L_LAYERS * 128))
    return out.reshape(T_DIM, N_NODES)
